# baseline (speedup 1.0000x reference)
import functools

import jax
import jax.numpy as jnp
from jax import lax
from jax.experimental import pallas as pl
from jax.experimental.pallas import tpu as pltpu
from jax.experimental.pallas import tpu_sc as plsc

N_ENTITIES = 1000000
N_RELATIONS = 1000
EMBED_DIM = 64
BATCH = 16384

NUM_CORES = 2
NUM_SUBCORES = 16
NUM_WORKERS = NUM_CORES * NUM_SUBCORES
B_PER_W = BATCH // NUM_WORKERS
LANES = 16
GROUPS = B_PER_W // LANES


def _neg_sqrt(x):
    i = lax.bitcast_convert_type(x, jnp.int32)
    i = 0x5F3759DF - lax.shift_right_arithmetic(i, 1)
    y = lax.bitcast_convert_type(i, jnp.float32)
    half_x = 0.5 * x
    y = y * (1.5 - half_x * y * y)
    y = y * (1.5 - half_x * y * y)
    y = y * (1.5 - half_x * y * y)
    return -(x * y)


def _sc_body(h_hbm, t_hbm, r_hbm, ent_hbm, rel_hbm, out_hbm,
             idx_h, idx_t, idx_r, rows_h, rows_t, rows_r, out_v, sem):
    wid = lax.axis_index("s") * NUM_CORES + lax.axis_index("c")
    base = wid * B_PER_W

    pltpu.sync_copy(h_hbm.at[pl.ds(base, B_PER_W)], idx_h)
    pltpu.sync_copy(t_hbm.at[pl.ds(base, B_PER_W)], idx_t)
    pltpu.sync_copy(r_hbm.at[pl.ds(base, B_PER_W)], idx_r)

    cp_h = pltpu.async_copy(ent_hbm.at[idx_h], rows_h, sem)
    cp_t = pltpu.async_copy(ent_hbm.at[idx_t], rows_t, sem)
    cp_r = pltpu.async_copy(rel_hbm.at[idx_r], rows_r, sem)
    cp_h.wait()
    cp_t.wait()
    cp_r.wait()

    lane = lax.iota(jnp.int32, LANES)

    def group_body(g, _):
        row_ids = g * LANES + lane
        acc = jnp.zeros((LANES,), jnp.float32)

        def dim_body(d, acc):
            col = jnp.full((LANES,), d, jnp.int32)
            vh = plsc.load_gather(rows_h, [row_ids, col])
            vr = plsc.load_gather(rows_r, [row_ids, col])
            vt = plsc.load_gather(rows_t, [row_ids, col])
            df = (vh + vr) - vt
            return acc + df * df

        acc = lax.fori_loop(0, EMBED_DIM, dim_body, acc)
        out_v[pl.ds(g * LANES, LANES)] = _neg_sqrt(acc + 1e-12)
        return _

    lax.fori_loop(0, GROUPS, group_body, 0)
    pltpu.sync_copy(out_v, out_hbm.at[pl.ds(base, B_PER_W)])


@jax.jit
def _score(h, t, r, entity_emb, relation_emb):
    mesh = plsc.VectorSubcoreMesh(core_axis_name="c", subcore_axis_name="s")
    run = functools.partial(
        pl.kernel,
        mesh=mesh,
        compiler_params=pltpu.CompilerParams(
            use_tc_tiling_on_sc=False, needs_layout_passes=False),
        out_type=jax.ShapeDtypeStruct((BATCH,), jnp.float32),
        scratch_types=[
            pltpu.VMEM((B_PER_W,), jnp.int32),
            pltpu.VMEM((B_PER_W,), jnp.int32),
            pltpu.VMEM((B_PER_W,), jnp.int32),
            pltpu.VMEM((B_PER_W, EMBED_DIM), jnp.float32),
            pltpu.VMEM((B_PER_W, EMBED_DIM), jnp.float32),
            pltpu.VMEM((B_PER_W, EMBED_DIM), jnp.float32),
            pltpu.VMEM((B_PER_W,), jnp.float32),
            pltpu.SemaphoreType.DMA,
        ],
    )(_sc_body)
    return run(h, t, r, entity_emb, relation_emb)


def kernel(h, t, r, entity_emb, relation_emb):
    return _score(h.astype(jnp.int32), t.astype(jnp.int32),
                  r.astype(jnp.int32), entity_emb, relation_emb)


# speedup vs baseline: 1.8666x; 1.8666x over previous
import functools

import jax
import jax.numpy as jnp
from jax import lax
from jax.experimental import pallas as pl
from jax.experimental.pallas import tpu as pltpu
from jax.experimental.pallas import tpu_sc as plsc

N_ENTITIES = 1000000
N_RELATIONS = 1000
EMBED_DIM = 64
BATCH = 16384

NUM_CORES = 2
NUM_SUBCORES = 16
NUM_WORKERS = NUM_CORES * NUM_SUBCORES
B_PER_W = BATCH // NUM_WORKERS
LANES = 16
CHUNK = 32
N_CHUNKS = B_PER_W // CHUNK
GROUPS_PER_CHUNK = CHUNK // LANES


def _neg_sqrt(x):
    i = lax.bitcast_convert_type(x, jnp.int32)
    i = 0x5F3759DF - lax.shift_right_arithmetic(i, 1)
    y = lax.bitcast_convert_type(i, jnp.float32)
    half_x = 0.5 * x
    y = y * (1.5 - half_x * y * y)
    y = y * (1.5 - half_x * y * y)
    y = y * (1.5 - half_x * y * y)
    return -(x * y)


def _sc_body(h_hbm, t_hbm, r_hbm, ent_hbm, rel_hbm, out_hbm,
             idx_h, idx_t, idx_r, tix_h, tix_t, tix_r,
             tiles_h, tiles_t, tiles_r, out_v, sem):
    wid = lax.axis_index("s") * NUM_CORES + lax.axis_index("c")
    base = wid * B_PER_W

    pltpu.sync_copy(h_hbm.at[pl.ds(base, B_PER_W)], idx_h)
    pltpu.sync_copy(t_hbm.at[pl.ds(base, B_PER_W)], idx_t)
    pltpu.sync_copy(r_hbm.at[pl.ds(base, B_PER_W)], idx_r)

    lane = lax.iota(jnp.int32, LANES)

    def tix_body(g, _):
        sl = pl.ds(g * LANES, LANES)
        tix_h[sl] = lax.shift_right_logical(idx_h[sl], 3)
        tix_t[sl] = lax.shift_right_logical(idx_t[sl], 3)
        tix_r[sl] = lax.shift_right_logical(idx_r[sl], 3)
        return _

    lax.fori_loop(0, B_PER_W // LANES, tix_body, 0)

    def chunk_body(c, _):
        copies = []
        for g in range(GROUPS_PER_CHUNK):
            gsl = pl.ds(c * CHUNK + g * LANES, LANES)
            vkh = tix_h[gsl]
            vkt = tix_t[gsl]
            vkr = tix_r[gsl]
            for j in range(LANES):
                k = g * LANES + j
                copies.append(pltpu.async_copy(
                    ent_hbm.at[vkh[j]], tiles_h.at[k], sem))
                copies.append(pltpu.async_copy(
                    ent_hbm.at[vkt[j]], tiles_t.at[k], sem))
                copies.append(pltpu.async_copy(
                    rel_hbm.at[vkr[j]], tiles_r.at[k], sem))
        for cp in copies:
            cp.wait()

        def group_body(g, _):
            k_vec = g * LANES + lane
            gsl = pl.ds(c * CHUNK + g * LANES, LANES)
            sub_h = lax.bitwise_and(idx_h[gsl], 7)
            sub_t = lax.bitwise_and(idx_t[gsl], 7)
            sub_r = lax.bitwise_and(idx_r[gsl], 7)
            acc = jnp.zeros((LANES,), jnp.float32)

            def dim_body(d, acc):
                col = jnp.full((LANES,), d, jnp.int32)
                vh = plsc.load_gather(tiles_h, [k_vec, sub_h, col])
                vr = plsc.load_gather(tiles_r, [k_vec, sub_r, col])
                vt = plsc.load_gather(tiles_t, [k_vec, sub_t, col])
                df = (vh + vr) - vt
                return acc + df * df

            acc = lax.fori_loop(0, EMBED_DIM, dim_body, acc)
            out_v[gsl] = _neg_sqrt(acc + 1e-12)
            return _

        lax.fori_loop(0, GROUPS_PER_CHUNK, group_body, 0)
        return _

    lax.fori_loop(0, N_CHUNKS, chunk_body, 0)

    pltpu.sync_copy(out_v, out_hbm.at[pl.ds(base, B_PER_W)])


@jax.jit
def _score(h, t, r, entity_emb, relation_emb):
    ent3 = entity_emb.reshape(N_ENTITIES // 8, 8, EMBED_DIM)
    rel3 = relation_emb.reshape(N_RELATIONS // 8, 8, EMBED_DIM)
    mesh = plsc.VectorSubcoreMesh(core_axis_name="c", subcore_axis_name="s")
    run = functools.partial(
        pl.kernel,
        mesh=mesh,
        compiler_params=pltpu.CompilerParams(needs_layout_passes=False),
        out_type=jax.ShapeDtypeStruct((BATCH,), jnp.float32),
        scratch_types=[
            pltpu.VMEM((B_PER_W,), jnp.int32),
            pltpu.VMEM((B_PER_W,), jnp.int32),
            pltpu.VMEM((B_PER_W,), jnp.int32),
            pltpu.VMEM((B_PER_W,), jnp.int32),
            pltpu.VMEM((B_PER_W,), jnp.int32),
            pltpu.VMEM((B_PER_W,), jnp.int32),
            pltpu.VMEM((CHUNK, 8, EMBED_DIM), jnp.float32),
            pltpu.VMEM((CHUNK, 8, EMBED_DIM), jnp.float32),
            pltpu.VMEM((CHUNK, 8, EMBED_DIM), jnp.float32),
            pltpu.VMEM((B_PER_W,), jnp.float32),
            pltpu.SemaphoreType.DMA,
        ],
    )(_sc_body)
    return run(h, t, r, ent3, rel3)


def kernel(h, t, r, entity_emb, relation_emb):
    return _score(h.astype(jnp.int32), t.astype(jnp.int32),
                  r.astype(jnp.int32), entity_emb, relation_emb)
